# Initial kernel scaffold; baseline (speedup 1.0000x reference)
import jax
import jax.numpy as jnp
from jax.experimental import pallas as pl


def kernel(fushed_features, input_data, centroids, Wc1a, bc1a, Wc1b, bc1b, Wc2, bc2, Wi1a, bi1a, Wi1b, bi1b, Wi2, bi2, Wg, bg, adj):
    raise NotImplementedError("write your pallas kernel here")


# speedup vs baseline: 13.4822x; 13.4822x over previous
import functools

import jax
import jax.numpy as jnp
from jax import lax
from jax.experimental import pallas as pl
from jax.experimental.pallas import tpu as pltpu
from jax.experimental.pallas import tpu_sc as plsc

B, N, K, SX, F, MID, DOUT, C, SO = 4, 10000, 16, 12, 32, 16, 16, 8, 32
MARGIN = 0.5
UPDATE_RATE = 0.01

R = B * N
CSO = C * SO
W = CSO + 2 * C
NCORES, NSUB = 2, 16
NW = NCORES * NSUB
CH = 128
NCH = 10
PER_TILE = CH * NCH
RP = NW * PER_TILE
BLK = 2000
NBLK = R // BLK


def _relu(x):
    return jnp.maximum(x, 0.0)


def _dot(a, b):
    return jnp.dot(a, b, preferred_element_type=jnp.float32)


def _expand_mat(rows, cols, group):
    r = lax.broadcasted_iota(jnp.int32, (rows, cols), 0)
    j = lax.broadcasted_iota(jnp.int32, (rows, cols), 1)
    return (j // group == r).astype(jnp.float32)


def _s1_body(fushed_ref, inp_ref, cent_ref, wc1a, bc1a, wc1b, bc1b, wc2, bc2,
             wi1a, bi1a, wi1b, bi1b, wi2, bi2, wgt, bgt, out_ref):
    x = fushed_ref[...]
    fused = _relu(_dot(_relu(_dot(x, wi1a[...]) + bi1a[...]), wi1b[...])
                  + bi1b[...]) + _relu(_dot(x, wi2[...]) + bi2[...])
    cent = cent_ref[...]
    cf = _relu(_dot(_relu(_dot(cent, wc1a[...]) + bc1a[...]), wc1b[...])
               + bc1b[...]) + _relu(_dot(cent, wc2[...]) + bc2[...])
    xn = jnp.sum(fused * fused, axis=1, keepdims=True)
    cn = lax.dot_general(jnp.ones((1, DOUT), jnp.float32), cf * cf,
                         (((1,), (1,)), ((), ())))
    g = lax.dot_general(fused, cf, (((1,), (1,)), ((), ())))
    dist = jnp.sqrt(jnp.maximum(xn + cn - 2.0 * g, 1e-30))
    m = jnp.max(dist, axis=1, keepdims=True)
    e = jnp.exp(dist - m)
    simi = e / jnp.sum(e, axis=1, keepdims=True)
    t = _relu(_dot(inp_ref[...], wgt[...]) + bgt[...])
    h = t * _dot(simi, _expand_mat(C, CSO, SO))
    out_ref[...] = jnp.concatenate(
        [h, simi, jnp.zeros((BLK, C), jnp.float32)], axis=1)


def _stage1(fushed, inp, cent, wc1a, bc1a, wc1b, bc1b, wc2, bc2,
            wi1a, bi1a, wi1b, bi1b, wi2, bi2, wgt, bgt):
    row_spec = lambda width: pl.BlockSpec((BLK, width), lambda i: (i, 0))
    rep = lambda shape: pl.BlockSpec(shape, lambda i: (0,) * len(shape))
    return pl.pallas_call(
        _s1_body,
        grid=(NBLK,),
        in_specs=[row_spec(F), row_spec(SX), rep((C, F)),
                  rep((F, MID)), rep((1, MID)), rep((MID, DOUT)), rep((1, DOUT)),
                  rep((F, DOUT)), rep((1, DOUT)),
                  rep((F, MID)), rep((1, MID)), rep((MID, DOUT)), rep((1, DOUT)),
                  rep((F, DOUT)), rep((1, DOUT)),
                  rep((SX, CSO)), rep((1, CSO))],
        out_specs=row_spec(W),
        out_shape=jax.ShapeDtypeStruct((R, W), jnp.float32),
    )(fushed, inp, cent, wc1a, bc1a, wc1b, bc1b, wc2, bc2,
      wi1a, bi1a, wi1b, bi1b, wi2, bi2, wgt, bgt)


def _s2_body(hs_hbm, adjt_hbm, out_hbm, idx_v, acc_v, sem0, sem1):
    wid = lax.axis_index("s") * NCORES + lax.axis_index("c")
    pltpu.sync_copy(adjt_hbm.at[:, wid], idx_v)

    def chunk(j, carry):
        pltpu.async_copy(hs_hbm.at[idx_v.at[0, j]], acc_v, sem0).wait()
        for k in range(1, K):
            pltpu.async_copy(hs_hbm.at[idx_v.at[k, j]], acc_v, sem1,
                             add=True).wait()
        pltpu.sync_copy(acc_v, out_hbm.at[pl.ds(wid * PER_TILE + j * CH, CH)])
        return carry

    lax.fori_loop(0, NCH, chunk, 0)


_stage2 = functools.partial(
    pl.kernel,
    out_type=jax.ShapeDtypeStruct((RP, W), jnp.float32),
    mesh=plsc.VectorSubcoreMesh(core_axis_name="c", subcore_axis_name="s",
                                num_cores=NCORES, num_subcores=NSUB),
    scratch_types=[pltpu.VMEM((K, NCH, CH), jnp.int32),
                   pltpu.VMEM((CH, W), jnp.float32),
                   pltpu.SemaphoreType.DMA,
                   pltpu.SemaphoreType.DMA],
    compiler_params=pltpu.CompilerParams(use_tc_tiling_on_sc=False),
)(_s2_body)


def _s3_body(nd_ref, upd_ref, sum_ref):
    nd = nd_ref[...]
    numer = nd[:, :CSO]
    denom = nd[:, CSO:CSO + C]
    upd = numer / _dot(denom, _expand_mat(C, CSO, SO))
    upd_ref[...] = upd
    part = jnp.sum(upd, axis=0, keepdims=True)

    @pl.when(pl.program_id(0) == 0)
    def _():
        sum_ref[...] = part

    @pl.when(pl.program_id(0) > 0)
    def _():
        sum_ref[...] = sum_ref[...] + part


def _stage3(nd):
    return pl.pallas_call(
        _s3_body,
        grid=(NBLK,),
        in_specs=[pl.BlockSpec((BLK, W), lambda i: (i, 0))],
        out_specs=[pl.BlockSpec((BLK, CSO), lambda i: (i, 0)),
                   pl.BlockSpec((1, CSO), lambda i: (0, 0))],
        out_shape=[jax.ShapeDtypeStruct((R, CSO), jnp.float32),
                   jax.ShapeDtypeStruct((1, CSO), jnp.float32)],
    )(nd)


def _s4_body(sum_ref, cent_ref, out_ref):
    mean = sum_ref[...] * (1.0 / R)
    nc = (1.0 - UPDATE_RATE) * cent_ref[...] + UPDATE_RATE * mean
    sq = nc * nc
    ones = jnp.ones((1, SO), jnp.float32)
    ncol = lax.dot_general(ones, sq, (((1,), (1,)), ((), ())))
    nrow = lax.dot_general(sq, ones, (((1,), (1,)), ((), ())))
    g = lax.dot_general(nc, nc, (((1,), (1,)), ((), ())))
    dist = jnp.sqrt(jnp.maximum(nrow + ncol - 2.0 * g, 1e-30))
    i = lax.broadcasted_iota(jnp.int32, (C, C), 0)
    j = lax.broadcasted_iota(jnp.int32, (C, C), 1)
    target = jnp.where(i == j, 0.0, MARGIN)
    out_ref[...] = jnp.sum(jnp.maximum(target - dist, 0.0) ** 2,
                           keepdims=True)


def _stage4(sums, cent):
    return pl.pallas_call(
        _s4_body,
        in_specs=[pl.BlockSpec((C, SO), lambda: (0, 0)),
                  pl.BlockSpec((C, SO), lambda: (0, 0))],
        out_specs=pl.BlockSpec((1, 1), lambda: (0, 0)),
        out_shape=jax.ShapeDtypeStruct((1, 1), jnp.float32),
    )(sums, cent)


def kernel(fushed_features, input_data, centroids, Wc1a, bc1a, Wc1b, bc1b,
           Wc2, bc2, Wi1a, bi1a, Wi1b, bi1b, Wi2, bi2, Wg, bg, adj):
    fushed = fushed_features.reshape(R, F)
    inp = input_data[:, 0].reshape(R, SX)
    wgt = jnp.transpose(Wg, (1, 0, 2)).reshape(SX, CSO)
    bgt = bg.reshape(1, CSO)
    r2 = lambda v: v.reshape(1, -1)

    hs = _stage1(fushed, inp, centroids,
                 Wc1a, r2(bc1a), Wc1b, r2(bc1b), Wc2, r2(bc2),
                 Wi1a, r2(bi1a), Wi1b, r2(bi1b), Wi2, r2(bi2), wgt, bgt)

    adjg = adj + (jnp.arange(B, dtype=jnp.int32) * N)[:, None, None]
    adjt = jnp.transpose(adjg, (2, 0, 1)).reshape(K, R)
    adjt = jnp.pad(adjt, ((0, 0), (0, RP - R))).reshape(K, NW, NCH, CH)

    nd = _stage2(hs, adjt)
    upd, sums = _stage3(nd)
    loss = _stage4(sums.reshape(C, SO), centroids)
    return upd.reshape(B, N, C, SO), loss[0, 0]


# speedup vs baseline: 15.4319x; 1.1446x over previous
import functools

import jax
import jax.numpy as jnp
from jax import lax
from jax.experimental import pallas as pl
from jax.experimental.pallas import tpu as pltpu
from jax.experimental.pallas import tpu_sc as plsc

B, N, K, SX, F, MID, DOUT, C, SO = 4, 10000, 16, 12, 32, 16, 16, 8, 32
MARGIN = 0.5
UPDATE_RATE = 0.01

R = B * N
CSO = C * SO
W = CSO + 2 * C
NCORES, NSUB = 2, 16
NW = NCORES * NSUB
CH = 128
NCH = 10
PER_TILE = CH * NCH
RP = NW * PER_TILE
BLK = 2000
NBLK = R // BLK


def _relu(x):
    return jnp.maximum(x, 0.0)


def _dot(a, b):
    return jnp.dot(a, b, preferred_element_type=jnp.float32)


def _expand_mat(rows, cols, group):
    r = lax.broadcasted_iota(jnp.int32, (rows, cols), 0)
    j = lax.broadcasted_iota(jnp.int32, (rows, cols), 1)
    return (j // group == r).astype(jnp.float32)


def _s1_body(fushed_ref, inp_ref, cent_ref, wc1a, bc1a, wc1b, bc1b, wc2, bc2,
             wi1a, bi1a, wi1b, bi1b, wi2, bi2, wgt, bgt, out_ref):
    x = fushed_ref[...]
    fused = _relu(_dot(_relu(_dot(x, wi1a[...]) + bi1a[...]), wi1b[...])
                  + bi1b[...]) + _relu(_dot(x, wi2[...]) + bi2[...])
    cent = cent_ref[...]
    cf = _relu(_dot(_relu(_dot(cent, wc1a[...]) + bc1a[...]), wc1b[...])
               + bc1b[...]) + _relu(_dot(cent, wc2[...]) + bc2[...])
    xn = jnp.sum(fused * fused, axis=1, keepdims=True)
    cn = lax.dot_general(jnp.ones((1, DOUT), jnp.float32), cf * cf,
                         (((1,), (1,)), ((), ())))
    g = lax.dot_general(fused, cf, (((1,), (1,)), ((), ())))
    dist = jnp.sqrt(jnp.maximum(xn + cn - 2.0 * g, 1e-30))
    m = jnp.max(dist, axis=1, keepdims=True)
    e = jnp.exp(dist - m)
    simi = e / jnp.sum(e, axis=1, keepdims=True)
    t = _relu(_dot(inp_ref[...], wgt[...]) + bgt[...])
    h = t * _dot(simi, _expand_mat(C, CSO, SO))
    out_ref[...] = jnp.concatenate(
        [h, simi, jnp.zeros((BLK, C), jnp.float32)], axis=1)


def _stage1(fushed, inp, cent, wc1a, bc1a, wc1b, bc1b, wc2, bc2,
            wi1a, bi1a, wi1b, bi1b, wi2, bi2, wgt, bgt):
    row_spec = lambda width: pl.BlockSpec((BLK, width), lambda i: (i, 0))
    rep = lambda shape: pl.BlockSpec(shape, lambda i: (0,) * len(shape))
    return pl.pallas_call(
        _s1_body,
        grid=(NBLK,),
        in_specs=[row_spec(F), row_spec(SX), rep((C, F)),
                  rep((F, MID)), rep((1, MID)), rep((MID, DOUT)), rep((1, DOUT)),
                  rep((F, DOUT)), rep((1, DOUT)),
                  rep((F, MID)), rep((1, MID)), rep((MID, DOUT)), rep((1, DOUT)),
                  rep((F, DOUT)), rep((1, DOUT)),
                  rep((SX, CSO)), rep((1, CSO))],
        out_specs=row_spec(W),
        out_shape=jax.ShapeDtypeStruct((R, W), jnp.float32),
    )(fushed, inp, cent, wc1a, bc1a, wc1b, bc1b, wc2, bc2,
      wi1a, bi1a, wi1b, bi1b, wi2, bi2, wgt, bgt)


def _s2_body(hs_hbm, adjt_hbm, out_hbm, idx_v, acc_v, sem0, sem1):
    wid = lax.axis_index("s") * NCORES + lax.axis_index("c")
    pltpu.sync_copy(adjt_hbm.at[:, wid], idx_v)

    def chunk(j, carry):
        pltpu.async_copy(hs_hbm.at[idx_v.at[0, j]], acc_v, sem0).wait()
        descs = [pltpu.async_copy(hs_hbm.at[idx_v.at[k, j]], acc_v, sem1,
                                  add=True) for k in range(1, K)]
        for d in descs:
            d.wait()
        pltpu.sync_copy(acc_v, out_hbm.at[pl.ds(wid * PER_TILE + j * CH, CH)])
        return carry

    lax.fori_loop(0, NCH, chunk, 0)


_stage2 = functools.partial(
    pl.kernel,
    out_type=jax.ShapeDtypeStruct((RP, W), jnp.float32),
    mesh=plsc.VectorSubcoreMesh(core_axis_name="c", subcore_axis_name="s",
                                num_cores=NCORES, num_subcores=NSUB),
    scratch_types=[pltpu.VMEM((K, NCH, CH), jnp.int32),
                   pltpu.VMEM((CH, W), jnp.float32),
                   pltpu.SemaphoreType.DMA,
                   pltpu.SemaphoreType.DMA],
    compiler_params=pltpu.CompilerParams(use_tc_tiling_on_sc=False),
)(_s2_body)


def _s3_body(nd_ref, upd_ref, sum_ref):
    nd = nd_ref[...]
    numer = nd[:, :CSO]
    denom = nd[:, CSO:CSO + C]
    upd = numer / _dot(denom, _expand_mat(C, CSO, SO))
    upd_ref[...] = upd
    part = jnp.sum(upd, axis=0, keepdims=True)

    @pl.when(pl.program_id(0) == 0)
    def _():
        sum_ref[...] = part

    @pl.when(pl.program_id(0) > 0)
    def _():
        sum_ref[...] = sum_ref[...] + part


def _stage3(nd):
    return pl.pallas_call(
        _s3_body,
        grid=(NBLK,),
        in_specs=[pl.BlockSpec((BLK, W), lambda i: (i, 0))],
        out_specs=[pl.BlockSpec((BLK, CSO), lambda i: (i, 0)),
                   pl.BlockSpec((1, CSO), lambda i: (0, 0))],
        out_shape=[jax.ShapeDtypeStruct((R, CSO), jnp.float32),
                   jax.ShapeDtypeStruct((1, CSO), jnp.float32)],
    )(nd)


def _s4_body(sum_ref, cent_ref, out_ref):
    mean = sum_ref[...] * (1.0 / R)
    nc = (1.0 - UPDATE_RATE) * cent_ref[...] + UPDATE_RATE * mean
    sq = nc * nc
    ones = jnp.ones((1, SO), jnp.float32)
    ncol = lax.dot_general(ones, sq, (((1,), (1,)), ((), ())))
    nrow = lax.dot_general(sq, ones, (((1,), (1,)), ((), ())))
    g = lax.dot_general(nc, nc, (((1,), (1,)), ((), ())))
    dist = jnp.sqrt(jnp.maximum(nrow + ncol - 2.0 * g, 1e-30))
    i = lax.broadcasted_iota(jnp.int32, (C, C), 0)
    j = lax.broadcasted_iota(jnp.int32, (C, C), 1)
    target = jnp.where(i == j, 0.0, MARGIN)
    out_ref[...] = jnp.sum(jnp.maximum(target - dist, 0.0) ** 2,
                           keepdims=True)


def _stage4(sums, cent):
    return pl.pallas_call(
        _s4_body,
        in_specs=[pl.BlockSpec((C, SO), lambda: (0, 0)),
                  pl.BlockSpec((C, SO), lambda: (0, 0))],
        out_specs=pl.BlockSpec((1, 1), lambda: (0, 0)),
        out_shape=jax.ShapeDtypeStruct((1, 1), jnp.float32),
    )(sums, cent)


def kernel(fushed_features, input_data, centroids, Wc1a, bc1a, Wc1b, bc1b,
           Wc2, bc2, Wi1a, bi1a, Wi1b, bi1b, Wi2, bi2, Wg, bg, adj):
    fushed = fushed_features.reshape(R, F)
    inp = input_data[:, 0].reshape(R, SX)
    wgt = jnp.transpose(Wg, (1, 0, 2)).reshape(SX, CSO)
    bgt = bg.reshape(1, CSO)
    r2 = lambda v: v.reshape(1, -1)

    hs = _stage1(fushed, inp, centroids,
                 Wc1a, r2(bc1a), Wc1b, r2(bc1b), Wc2, r2(bc2),
                 Wi1a, r2(bi1a), Wi1b, r2(bi1b), Wi2, r2(bi2), wgt, bgt)

    adjg = adj + (jnp.arange(B, dtype=jnp.int32) * N)[:, None, None]
    adjt = jnp.transpose(adjg, (2, 0, 1)).reshape(K, R)
    adjt = jnp.pad(adjt, ((0, 0), (0, RP - R))).reshape(K, NW, NCH, CH)

    nd = _stage2(hs, adjt)
    upd, sums = _stage3(nd)
    loss = _stage4(sums.reshape(C, SO), centroids)
    return upd.reshape(B, N, C, SO), loss[0, 0]
